# baseline (speedup 1.0000x reference)
import functools

import jax
import jax.numpy as jnp
from jax import lax
from jax.experimental import pallas as pl
from jax.experimental.pallas import tpu as pltpu
from jax.experimental.pallas import tpu_sc as plsc

_WINDOW = 200
_NUM_CORES = 2
_NUM_SUBCORES = 16
_NW = _NUM_CORES * _NUM_SUBCORES
_LANES = 16
_BB = 128


@functools.partial(jax.jit, static_argnums=(2, 3, 4, 5))
def _sc_window_gather(tbl2, start, batch, b_per_w, k0, kchunks):
    _, half_rows, lanes = tbl2.shape
    mesh = plsc.VectorSubcoreMesh(
        core_axis_name="c",
        subcore_axis_name="s",
        num_cores=_NUM_CORES,
        num_subcores=_NUM_SUBCORES,
    )

    @functools.partial(
        pl.kernel,
        mesh=mesh,
        out_type=jax.ShapeDtypeStruct((kchunks, batch, lanes), jnp.float32),
        scratch_types=[
            pltpu.VMEM((b_per_w,), jnp.int32),
            pltpu.VMEM_SHARED((2, half_rows, lanes), jnp.float32),
            pltpu.SemaphoreType.DMA,
            pltpu.SemaphoreType.DMA,
        ],
        compiler_params=pltpu.CompilerParams(use_tc_tiling_on_sc=False),
    )
    def k(tbl_hbm, start_hbm, out_hbm, idx_v, tbl_sp, sem_idx, sem_out):
        sid = lax.axis_index("s")
        wid = sid * _NUM_CORES + lax.axis_index("c")
        base = wid * b_per_w
        idx_cp = pltpu.make_async_copy(
            start_hbm.at[pl.ds(base, b_per_w)], idx_v, sem_idx
        )
        idx_cp.start()

        @pl.when(sid == 0)
        def _():
            pltpu.make_async_copy(tbl_hbm, tbl_sp, sem_out).start()
            pltpu.make_async_copy(tbl_hbm, tbl_sp, sem_out).wait()

        idx_cp.wait()
        plsc.subcore_barrier()

        def fire(g, carry):
            vec = idx_v[pl.ds(g * _LANES, _LANES)]
            parity = lax.rem(vec, 2)
            row = lax.shift_right_logical(vec, 1) + k0
            for lane in range(_LANES):
                pltpu.make_async_copy(
                    tbl_sp.at[parity[lane], pl.ds(row[lane], kchunks), :],
                    out_hbm.at[:, base + g * _LANES + lane, :],
                    sem_out,
                ).start()
            return carry

        lax.fori_loop(0, b_per_w // _LANES, fire, 0)

        def drain(b, carry):
            pltpu.make_async_copy(
                tbl_sp.at[0, pl.ds(0, kchunks), :],
                out_hbm.at[:, base + b, :],
                sem_out,
            ).wait()
            return carry

        lax.fori_loop(0, b_per_w, drain, 0)

    return k(tbl2, start)


@functools.partial(jax.jit, static_argnums=(2, 3, 4))
def _tc_transpose(m, prev, batch, k0, ktotal):
    kchunks = m.shape[0]
    rb = 20

    def body(*refs):
        in_ref, out_ref = refs[-2], refs[-1]
        for k in range(rb):
            out_ref[k * _BB : (k + 1) * _BB, :] = in_ref[k].T

    m_spec = pl.BlockSpec((rb, _BB, _BB), lambda i, j: (j, i, 0))
    operands = (m,) if prev is None else (prev, m)
    in_specs = [m_spec] if prev is None else [
        pl.BlockSpec(memory_space=pl.ANY),
        m_spec,
    ]
    return pl.pallas_call(
        body,
        grid=(batch // _BB, kchunks // rb),
        in_specs=in_specs,
        out_specs=pl.BlockSpec(
            (rb * _BB, _BB), lambda i, j, _b=k0 // rb: (_b + j, i)
        ),
        out_shape=jax.ShapeDtypeStruct((ktotal * _BB, batch), jnp.float32),
        input_output_aliases={} if prev is None else {0: 0},
    )(*operands)


def kernel(index, length, data):
    cycle_len, channels = data.shape
    batch = index.shape[0]
    start = jnp.asarray(
        (index.astype(jnp.int32) + (length - _WINDOW)) % cycle_len, jnp.int32
    )
    flat = jnp.concatenate([data, data[: _WINDOW + 1]], axis=0).reshape(-1)
    half_words = (cycle_len // 2 + _WINDOW // 2) * 2 * channels
    tbl2 = jnp.stack(
        [
            flat[:half_words].reshape(-1, 2 * channels),
            flat[channels : half_words + channels].reshape(-1, 2 * channels),
        ]
    )
    phase_sizes = (40, 60)
    ktotal = sum(phase_sizes)
    outT, k0 = None, 0
    for kchunks in phase_sizes:
        m = _sc_window_gather(tbl2, start, batch, batch // _NW, k0, kchunks)
        outT = _tc_transpose(m, outT, batch, k0, ktotal)
        k0 += kchunks
    return outT.reshape(_WINDOW, channels, batch).transpose(2, 0, 1)


# speedup vs baseline: 1.0996x; 1.0996x over previous
import functools

import jax
import jax.numpy as jnp
from jax import lax
from jax.experimental import pallas as pl
from jax.experimental.pallas import tpu as pltpu
from jax.experimental.pallas import tpu_sc as plsc

_WINDOW = 200
_NUM_CORES = 2
_NUM_SUBCORES = 16
_NW = _NUM_CORES * _NUM_SUBCORES
_LANES = 16
_BB = 128


@functools.partial(jax.jit, static_argnums=(2, 3, 4, 5))
def _sc_window_gather(tbl2, start, batch, b_per_w, phase, nphases):
    _, half_rows, lanes = tbl2.shape
    kchunks = _WINDOW * 64 // lanes // nphases
    mesh = plsc.VectorSubcoreMesh(
        core_axis_name="c",
        subcore_axis_name="s",
        num_cores=_NUM_CORES,
        num_subcores=_NUM_SUBCORES,
    )

    @functools.partial(
        pl.kernel,
        mesh=mesh,
        out_type=jax.ShapeDtypeStruct((kchunks, batch, lanes), jnp.float32),
        scratch_types=[
            pltpu.VMEM((b_per_w,), jnp.int32),
            pltpu.VMEM_SHARED((2, half_rows, lanes), jnp.float32),
            pltpu.SemaphoreType.DMA,
            pltpu.SemaphoreType.DMA,
        ],
        compiler_params=pltpu.CompilerParams(use_tc_tiling_on_sc=False),
    )
    def k(tbl_hbm, start_hbm, out_hbm, idx_v, tbl_sp, sem_idx, sem_out):
        sid = lax.axis_index("s")
        wid = sid * _NUM_CORES + lax.axis_index("c")
        base = wid * b_per_w
        idx_cp = pltpu.make_async_copy(
            start_hbm.at[pl.ds(base, b_per_w)], idx_v, sem_idx
        )
        idx_cp.start()

        @pl.when(sid == 0)
        def _():
            pltpu.make_async_copy(tbl_hbm, tbl_sp, sem_out).start()
            pltpu.make_async_copy(tbl_hbm, tbl_sp, sem_out).wait()

        idx_cp.wait()
        plsc.subcore_barrier()

        def fire(g, carry):
            vec = idx_v[pl.ds(g * _LANES, _LANES)]
            parity = lax.rem(vec, 2)
            row = lax.shift_right_logical(vec, 1) + phase * kchunks
            for lane in range(_LANES):
                pltpu.make_async_copy(
                    tbl_sp.at[parity[lane], pl.ds(row[lane], kchunks), :],
                    out_hbm.at[:, base + g * _LANES + lane, :],
                    sem_out,
                ).start()
            return carry

        lax.fori_loop(0, b_per_w // _LANES, fire, 0)

        def drain(b, carry):
            pltpu.make_async_copy(
                tbl_sp.at[0, pl.ds(0, kchunks), :],
                out_hbm.at[:, base + b, :],
                sem_out,
            ).wait()
            return carry

        lax.fori_loop(0, b_per_w, drain, 0)

    return k(tbl2, start)


@functools.partial(jax.jit, static_argnums=(2, 3, 4))
def _tc_transpose(m, prev, batch, phase, nphases):
    kchunks = m.shape[0]
    rows = kchunks * _BB * nphases

    def body(*refs):
        in_ref, out_ref = refs[-2], refs[-1]
        for k in range(kchunks):
            out_ref[k * _BB : (k + 1) * _BB, :] = in_ref[k].T

    m_spec = pl.BlockSpec((kchunks, _BB, _BB), lambda i: (0, i, 0))
    operands = (m,) if prev is None else (prev, m)
    in_specs = [m_spec] if prev is None else [
        pl.BlockSpec(memory_space=pl.ANY),
        m_spec,
    ]
    return pl.pallas_call(
        body,
        grid=(batch // _BB,),
        in_specs=in_specs,
        out_specs=pl.BlockSpec(
            (kchunks * _BB, _BB), lambda i, _p=phase: (_p, i)
        ),
        out_shape=jax.ShapeDtypeStruct((rows, batch), jnp.float32),
        input_output_aliases={} if prev is None else {0: 0},
    )(*operands)


def kernel(index, length, data):
    cycle_len, channels = data.shape
    batch = index.shape[0]
    start = jnp.asarray(
        (index.astype(jnp.int32) + (length - _WINDOW)) % cycle_len, jnp.int32
    )
    flat = jnp.concatenate([data, data[: _WINDOW + 1]], axis=0).reshape(-1)
    half_words = (cycle_len // 2 + _WINDOW // 2) * 2 * channels
    tbl2 = jnp.stack(
        [
            flat[:half_words].reshape(-1, 2 * channels),
            flat[channels : half_words + channels].reshape(-1, 2 * channels),
        ]
    )
    nphases = 4
    outT = None
    for phase in range(nphases):
        m = _sc_window_gather(tbl2, start, batch, batch // _NW, phase, nphases)
        outT = _tc_transpose(m, outT, batch, phase, nphases)
    return outT.reshape(_WINDOW, channels, batch).transpose(2, 0, 1)


# speedup vs baseline: 1.1483x; 1.0443x over previous
import functools

import jax
import jax.numpy as jnp
from jax import lax
from jax.experimental import pallas as pl
from jax.experimental.pallas import tpu as pltpu
from jax.experimental.pallas import tpu_sc as plsc

_WINDOW = 200
_NUM_CORES = 2
_NUM_SUBCORES = 16
_NW = _NUM_CORES * _NUM_SUBCORES
_LANES = 16
_BB = 128


@functools.partial(jax.jit, static_argnums=(2, 3, 4, 5))
def _sc_window_gather(tbl2, start, batch, b_per_w, phase, nphases):
    _, half_rows, lanes = tbl2.shape
    kchunks = _WINDOW * 64 // lanes // nphases
    mesh = plsc.VectorSubcoreMesh(
        core_axis_name="c",
        subcore_axis_name="s",
        num_cores=_NUM_CORES,
        num_subcores=_NUM_SUBCORES,
    )

    @functools.partial(
        pl.kernel,
        mesh=mesh,
        out_type=jax.ShapeDtypeStruct((kchunks, batch, lanes), jnp.float32),
        scratch_types=[
            pltpu.VMEM((b_per_w,), jnp.int32),
            pltpu.VMEM_SHARED((2, half_rows, lanes), jnp.float32),
            pltpu.SemaphoreType.DMA,
            pltpu.SemaphoreType.DMA,
        ],
        compiler_params=pltpu.CompilerParams(use_tc_tiling_on_sc=False),
    )
    def k(tbl_hbm, start_hbm, out_hbm, idx_v, tbl_sp, sem_idx, sem_out):
        sid = lax.axis_index("s")
        wid = sid * _NUM_CORES + lax.axis_index("c")
        base = wid * b_per_w
        idx_cp = pltpu.make_async_copy(
            start_hbm.at[pl.ds(base, b_per_w)], idx_v, sem_idx
        )
        idx_cp.start()

        @pl.when(sid == 0)
        def _():
            pltpu.make_async_copy(tbl_hbm, tbl_sp, sem_out).start()
            pltpu.make_async_copy(tbl_hbm, tbl_sp, sem_out).wait()

        idx_cp.wait()
        plsc.subcore_barrier()

        def fire(g, carry):
            vec = idx_v[pl.ds(g * _LANES, _LANES)]
            parity = lax.rem(vec, 2)
            row = lax.shift_right_logical(vec, 1) + phase * kchunks
            for lane in range(_LANES):
                pltpu.make_async_copy(
                    tbl_sp.at[parity[lane], pl.ds(row[lane], kchunks), :],
                    out_hbm.at[:, base + g * _LANES + lane, :],
                    sem_out,
                ).start()
            return carry

        lax.fori_loop(0, b_per_w // _LANES, fire, 0)

        def drain(b, carry):
            pltpu.make_async_copy(
                tbl_sp.at[0, pl.ds(0, kchunks), :],
                out_hbm.at[:, base + b, :],
                sem_out,
            ).wait()
            return carry

        lax.fori_loop(0, b_per_w, drain, 0)

    return k(tbl2, start)


@functools.partial(jax.jit, static_argnums=(2, 3, 4))
def _tc_transpose(m, prev, batch, phase, nphases):
    kchunks = m.shape[0]
    rows = kchunks * _BB * nphases

    def body(*refs):
        in_ref, out_ref = refs[-2], refs[-1]
        for k in range(kchunks):
            out_ref[k * _BB : (k + 1) * _BB, :] = in_ref[k].T

    m_spec = pl.BlockSpec((kchunks, _BB, _BB), lambda i: (0, i, 0))
    operands = (m,) if prev is None else (prev, m)
    in_specs = [m_spec] if prev is None else [
        pl.BlockSpec(memory_space=pl.ANY),
        m_spec,
    ]
    return pl.pallas_call(
        body,
        grid=(batch // _BB,),
        in_specs=in_specs,
        out_specs=pl.BlockSpec(
            (kchunks * _BB, _BB), lambda i, _p=phase: (_p, i)
        ),
        out_shape=jax.ShapeDtypeStruct((rows, batch), jnp.float32),
        input_output_aliases={} if prev is None else {0: 0},
    )(*operands)


def kernel(index, length, data):
    cycle_len, channels = data.shape
    batch = index.shape[0]
    start = jnp.asarray(
        (index.astype(jnp.int32) + (length - _WINDOW)) % cycle_len, jnp.int32
    )
    flat = jnp.concatenate([data, data[: _WINDOW + 1]], axis=0).reshape(-1)
    half_words = (cycle_len // 2 + _WINDOW // 2) * 2 * channels
    tbl2 = jnp.stack(
        [
            flat[:half_words].reshape(-1, 2 * channels),
            flat[channels : half_words + channels].reshape(-1, 2 * channels),
        ]
    )
    nphases = 2
    outT = None
    for phase in range(nphases):
        m = _sc_window_gather(tbl2, start, batch, batch // _NW, phase, nphases)
        outT = _tc_transpose(m, outT, batch, phase, nphases)
    return outT.reshape(_WINDOW, channels, batch).transpose(2, 0, 1)
